# initial kernel scaffold (unmeasured)
import jax
import jax.numpy as jnp
from jax import lax
from jax.experimental import pallas as pl
from jax.experimental.pallas import tpu as pltpu

N_DEV = 32
M_PER = 128
N_PER = 64


def kernel(x, w_mat, scale_x, scale_w):
    m_per, k = x.shape
    _, n = w_mat.shape

    def body(x_ref, w_ref, sx_ref, sw_ref, out_ref,
             comm_ref, recv_ref, send_sems, recv_sems):
        my = lax.axis_index("i")

        acc = lax.dot_general(
            x_ref[:, :], w_ref[:, :],
            dimension_numbers=(((1,), (0,)), ((), ())),
            preferred_element_type=jnp.int32,
        )
        scale = sx_ref[0] * sw_ref[0]
        y = jnp.maximum(acc.astype(jnp.float32) * scale, 0.0)

        for j in range(N_DEV):
            comm_ref[j, :, :] = y[:, j * N_PER:(j + 1) * N_PER]

        recv_ref[my, :, :] = comm_ref[my, :, :]

        rdmas = []
        for d in range(1, N_DEV):
            tgt = lax.rem(my + d, N_DEV)
            rdma = pltpu.make_async_remote_copy(
                src_ref=comm_ref.at[tgt],
                dst_ref=recv_ref.at[my],
                send_sem=send_sems.at[d],
                recv_sem=recv_sems.at[d],
                device_id=(tgt,),
                device_id_type=pl.DeviceIdType.MESH,
            )
            rdma.start()
            rdmas.append(rdma)

        for rdma in rdmas:
            rdma.wait()

        out_ref[:, :] = recv_ref[:].reshape(N_DEV * m_per, N_PER)

    return pl.pallas_call(
        body,
        out_shape=jax.ShapeDtypeStruct((N_DEV * m_per, N_PER), jnp.float32),
        in_specs=[
            pl.BlockSpec(memory_space=pltpu.VMEM),
            pl.BlockSpec(memory_space=pltpu.VMEM),
            pl.BlockSpec(memory_space=pltpu.SMEM),
            pl.BlockSpec(memory_space=pltpu.SMEM),
        ],
        out_specs=pl.BlockSpec(memory_space=pltpu.VMEM),
        scratch_shapes=[
            pltpu.VMEM((N_DEV, M_PER, N_PER), jnp.float32),
            pltpu.VMEM((N_DEV, M_PER, N_PER), jnp.float32),
            pltpu.SemaphoreType.DMA((N_DEV,)),
            pltpu.SemaphoreType.DMA((N_DEV,)),
        ],
        compiler_params=pltpu.CompilerParams(collective_id=0),
    )(x, w_mat, scale_x, scale_w)


# baseline (device time: 45341 ns/iter reference)
import jax
import jax.numpy as jnp
from jax import lax
from jax.experimental import pallas as pl
from jax.experimental.pallas import tpu as pltpu

N_DEV = 32
M_PER = 128
N_PER = 64


def kernel(x, w_mat, scale_x, scale_w):
    m_per, k = x.shape
    _, n = w_mat.shape

    def body(x_ref, w_ref, sx_ref, sw_ref, out_ref,
             comm_ref, recv_ref, send_sems, recv_sems):
        my = lax.axis_index("i")

        acc = lax.dot_general(
            x_ref[:, :], w_ref[:, :],
            dimension_numbers=(((1,), (0,)), ((), ())),
            preferred_element_type=jnp.int32,
        )
        scale = sx_ref[0] * sw_ref[0]
        y = jnp.maximum(acc.astype(jnp.float32) * scale, 0.0)

        for j in range(N_DEV):
            comm_ref[j, :, :] = y[:, j * N_PER:(j + 1) * N_PER]

        recv_ref[my, :, :] = comm_ref[my, :, :]

        rdmas = []
        for d in range(1, N_DEV):
            tgt = lax.rem(my + d, N_DEV)
            rdma = pltpu.make_async_remote_copy(
                src_ref=comm_ref.at[tgt],
                dst_ref=recv_ref.at[my],
                send_sem=send_sems.at[d],
                recv_sem=recv_sems.at[d],
                device_id=(tgt,),
                device_id_type=pl.DeviceIdType.MESH,
            )
            rdma.start()
            rdmas.append(rdma)

        for rdma in rdmas:
            rdma.wait()

        out_ref[:, :] = recv_ref[:].reshape(N_DEV * m_per, N_PER)

    return pl.pallas_call(
        body,
        out_shape=jax.ShapeDtypeStruct((N_DEV * m_per, N_PER), jnp.float32),
        in_specs=[
            pl.BlockSpec(memory_space=pltpu.VMEM),
            pl.BlockSpec(memory_space=pltpu.VMEM),
            pl.BlockSpec(memory_space=pltpu.SMEM),
            pl.BlockSpec(memory_space=pltpu.SMEM),
        ],
        out_specs=pl.BlockSpec(memory_space=pltpu.VMEM),
        scratch_shapes=[
            pltpu.VMEM((N_DEV, M_PER, N_PER), jnp.float32),
            pltpu.VMEM((N_DEV, M_PER, N_PER), jnp.float32),
            pltpu.SemaphoreType.DMA((N_DEV,)),
            pltpu.SemaphoreType.DMA((N_DEV,)),
        ],
    )(x, w_mat, scale_x, scale_w)
